# Initial kernel scaffold; baseline (speedup 1.0000x reference)
import jax
import jax.numpy as jnp
from jax.experimental import pallas as pl


def kernel(pos, k):
    N = pos.shape[0]
    K = 16
    sq = jnp.sum(pos * pos, axis=1)
    chunk = 2000
    idx_chunks = []
    for s in range(0, N, chunk):
        q = pos[s:s + chunk]
        qsq = sq[s:s + chunk]
        g = (q[:, 0:1] * pos[:, 0].reshape(1, N)
             + q[:, 1:2] * pos[:, 1].reshape(1, N)
             + q[:, 2:3] * pos[:, 2].reshape(1, N))
        d = qsq[:, None] + sq[None, :] - 2.0 * g
        _, idx = jax.lax.top_k(-d, K)
        idx_chunks.append(idx)
    nbr_idx = jnp.concatenate(idx_chunks, axis=0)
    k_arr = jnp.asarray(k)
    nbr_idx = nbr_idx + (k_arr - k_arr).astype(nbr_idx.dtype)
    center_idx = jnp.broadcast_to(jnp.arange(N, dtype=nbr_idx.dtype)[:, None], (N, K))
    return (nbr_idx.reshape(-1), center_idx.reshape(-1))


# speedup vs baseline: 3.1207x; 3.1207x over previous
import functools

import jax
import jax.numpy as jnp
from jax.experimental import pallas as pl
from jax.experimental.pallas import tpu as pltpu

_N = 20000
_K = 16
_TQ = 128
_NPAD = 20096
_GRID = _NPAD // _TQ


def _knn_body(xb, yb, zb, sqv, qx, qy, qz, qsq, out_ref):
    x = xb[...].astype(jnp.float32)
    y = yb[...].astype(jnp.float32)
    z = zb[...].astype(jnp.float32)
    s = sqv[...]
    ax = qx[...].astype(jnp.float32)
    ay = qy[...].astype(jnp.float32)
    az = qz[...].astype(jnp.float32)
    aq = qsq[...]
    g = ax * x + ay * y + az * z
    d = (aq + s) - 2.0 * g
    col = jax.lax.broadcasted_iota(jnp.int32, (_TQ, _NPAD), 1)
    big_i = jnp.int32(_NPAD)
    inf = jnp.float32(jnp.inf)
    for t in range(_K):
        m = jnp.min(d, axis=1, keepdims=True)
        jsel = jnp.min(jnp.where(d == m, col, big_i), axis=1, keepdims=True)
        out_ref[:, t:t + 1] = jsel
        d = jnp.where(col == jsel, inf, d)


def _knn(xb, yb, zb, sqv, qx, qy, qz, qsq):
    keyspec = pl.BlockSpec((1, _NPAD), lambda i: (0, 0))
    qspec = pl.BlockSpec((_TQ, 1), lambda i: (i, 0))
    return pl.pallas_call(
        _knn_body,
        grid=(_GRID,),
        in_specs=[keyspec, keyspec, keyspec, keyspec, qspec, qspec, qspec, qspec],
        out_specs=pl.BlockSpec((_TQ, _K), lambda i: (i, 0)),
        out_shape=jax.ShapeDtypeStruct((_NPAD, _K), jnp.int32),
        compiler_params=pltpu.CompilerParams(
            dimension_semantics=("arbitrary",),
        ),
    )(xb, yb, zb, sqv, qx, qy, qz, qsq)


def kernel(pos, k):
    n = pos.shape[0]
    sq = jnp.sum(pos * pos, axis=1)
    pad = _NPAD - n
    xf = jnp.concatenate([pos[:, 0], jnp.full((pad,), 1e6, jnp.float32)])
    yf = jnp.concatenate([pos[:, 1], jnp.full((pad,), 1e6, jnp.float32)])
    zf = jnp.concatenate([pos[:, 2], jnp.full((pad,), 1e6, jnp.float32)])
    sqp = jnp.concatenate([sq, jnp.full((pad,), 3e12, jnp.float32)])
    xb = xf.astype(jnp.bfloat16)
    yb = yf.astype(jnp.bfloat16)
    zb = zf.astype(jnp.bfloat16)
    idx = _knn(xb.reshape(1, _NPAD), yb.reshape(1, _NPAD), zb.reshape(1, _NPAD),
               sqp.reshape(1, _NPAD),
               xb.reshape(_NPAD, 1), yb.reshape(_NPAD, 1), zb.reshape(_NPAD, 1),
               sqp.reshape(_NPAD, 1))
    nbr = idx[:n]
    k_arr = jnp.asarray(k)
    nbr = nbr + (k_arr - k_arr).astype(nbr.dtype)
    center = jnp.broadcast_to(jnp.arange(n, dtype=nbr.dtype)[:, None], (n, _K))
    return (nbr.reshape(-1), center.reshape(-1))
